# speedup vs baseline: 2.9702x; 2.9702x over previous
import jax
import jax.numpy as jnp
from jax.experimental import pallas as pl

NUM_CLASSES = 80
MAX_BOXES = 100
N = 60


def _label_kernel(label_ref, box8_ref, obj8_ref, cls8_ref,
                  box16_ref, obj16_ref, cls16_ref,
                  box32_ref, obj32_ref, cls32_ref):
    lab = label_ref[...]
    cls_f = lab[:, 0]
    x = lab[:, 1]
    y = lab[:, 2]
    wh = lab[:, 3:5]

    c8x = (x * 0.125).astype(jnp.int32)
    c8y = (y * 0.125).astype(jnp.int32)
    c16x = jax.lax.shift_right_logical(c8x, 4)
    c16y = jax.lax.shift_right_logical(c8y, 4)
    c32x = jax.lax.shift_right_logical(c16x, 5)
    c32y = jax.lax.shift_right_logical(c16y, 5)
    cls_i = cls_f.astype(jnp.int32)

    pad = jnp.zeros((MAX_BOXES - N, 4), dtype=jnp.float32)
    box8 = jnp.concatenate([lab[:, 1:5], pad], axis=0)
    box16 = jnp.concatenate(
        [jnp.stack([c8x.astype(jnp.float32), c8y.astype(jnp.float32)], axis=1),
         wh], axis=1)
    box16 = jnp.concatenate([box16, pad], axis=0)
    box32 = jnp.concatenate(
        [jnp.stack([c16x.astype(jnp.float32), c16y.astype(jnp.float32)], axis=1),
         wh], axis=1)
    box32 = jnp.concatenate([box32, pad], axis=0)
    box8_ref[...] = box8
    box16_ref[...] = box16
    box32_ref[...] = box32

    ohc = (cls_i[:, None] == jax.lax.broadcasted_iota(jnp.int32, (N, NUM_CLASSES), 1)
           ).astype(jnp.float32)

    def grids(cx, cy, n_cells, obj_ref, cls_ref):
        ohx = (jax.lax.broadcasted_iota(jnp.int32, (n_cells, N), 0) == cx[None, :]
               ).astype(jnp.float32)
        ohy = (cy[:, None] == jax.lax.broadcasted_iota(jnp.int32, (N, n_cells), 1)
               ).astype(jnp.float32)
        obj_ref[...] = jax.lax.dot(ohx, ohy,
                                   preferred_element_type=jnp.float32)
        lin = cx * n_cells + cy
        ohcell = (jax.lax.broadcasted_iota(jnp.int32, (n_cells * n_cells, N), 0)
                  == lin[None, :]).astype(jnp.float32)
        cls_ref[...] = jax.lax.dot(
            ohcell, ohc, preferred_element_type=jnp.float32
        ).reshape(n_cells, n_cells, NUM_CLASSES)

    grids(c8x, c8y, 64, obj8_ref, cls8_ref)
    grids(c16x, c16y, 32, obj16_ref, cls16_ref)
    grids(c32x, c32y, 16, obj32_ref, cls32_ref)


def _f32(*shape):
    return jax.ShapeDtypeStruct(shape, jnp.float32)


def kernel(image, label):
    outs = pl.pallas_call(
        _label_kernel,
        out_shape=(
            _f32(MAX_BOXES, 4), _f32(64, 64), _f32(64, 64, NUM_CLASSES),
            _f32(MAX_BOXES, 4), _f32(32, 32), _f32(32, 32, NUM_CLASSES),
            _f32(MAX_BOXES, 4), _f32(16, 16), _f32(16, 16, NUM_CLASSES),
        ),
    )(label)
    (box8, obj8, cls8, box16, obj16, cls16, box32, obj32, cls32) = outs
    return (image, box8, obj8, cls8, box16, obj16, cls16,
            box32, obj32, cls32)
